# baseline (speedup 1.0000x reference)
import jax
import jax.numpy as jnp
from jax import lax
from jax.experimental import pallas as pl
from jax.experimental.pallas import tpu as pltpu
from jax.experimental.pallas import tpu_sc as plsc

N = 10000
D = 128
H = 128
P = 100000
E = 320000

_NC = 2
_NS = 16
_NW = _NC * _NS

_W = 64
_KB = 4
_SB = 40
_ECHUNKS = _KB * _SB
_EPAD = _NW * _ECHUNKS * _W

_NPAD = 10008
_ZR = 624

_PW = 128
_PCHUNKS = 26
_PPAD = _NW * _PCHUNKS * _PW

_ROW_BLK = 2000
_MLP_BLK = 2000


def _proj_body(x_ref, wk, bk, wq, bq, wv, bv, ws, bs,
               kn_ref, qv_ref, s_ref):
    xb = x_ref[...]
    kn_ref[...] = -(jnp.dot(xb, wk[...]) + bk[...])
    qv_ref[...] = jnp.concatenate(
        [jnp.dot(xb, wq[...]) + bq[...], jnp.dot(xb, wv[...]) + bv[...]],
        axis=1)
    s_ref[...] = jnp.dot(xb, ws[...]) + bs[...]


def _proj(x, Wk, bk, Wq, bq, Wv, bv, Ws, bs):
    nblk = N // _ROW_BLK
    row = pl.BlockSpec((_ROW_BLK, H), lambda i: (i, 0))
    wspec = pl.BlockSpec((H, H), lambda i: (0, 0))
    bspec = pl.BlockSpec((1, H), lambda i: (0, 0))
    row2 = pl.BlockSpec((_ROW_BLK, 2 * H), lambda i: (i, 0))
    out = jax.ShapeDtypeStruct((_NPAD, H), jnp.float32)
    out2 = jax.ShapeDtypeStruct((_NPAD, 2 * H), jnp.float32)
    return pl.pallas_call(
        _proj_body,
        grid=(nblk,),
        in_specs=[row, wspec, bspec, wspec, bspec, wspec, bspec, wspec, bspec],
        out_specs=[row, row2, row],
        out_shape=[out, out2, out],
    )(x, Wk, bk.reshape(1, H), Wq, bq.reshape(1, H), Wv, bv.reshape(1, H),
      Ws, bs.reshape(1, H))


def _combine_norm_proj_body(s_ref, pa_ref, pb_ref, wk, bk, wq, bq, wv, bv,
                            ws, bs, kn_ref, qv_ref, s2_ref):
    h = s_ref[...] + pa_ref[0] + pb_ref[0]
    nrm = jnp.sqrt(jnp.sum(h * h, axis=1, keepdims=True))
    h = h / jnp.maximum(nrm, 1e-12)
    h = jnp.maximum(h, 0.0)
    kn_ref[...] = -(jnp.dot(h, wk[...]) + bk[...])
    qv_ref[...] = jnp.concatenate(
        [jnp.dot(h, wq[...]) + bq[...], jnp.dot(h, wv[...]) + bv[...]],
        axis=1)
    s2_ref[...] = jnp.dot(h, ws[...]) + bs[...]


def _combine_norm_proj(s1, partials, Wk, bk, Wq, bq, Wv, bv, Ws, bs):
    nblk = N // _ROW_BLK
    row = pl.BlockSpec((_ROW_BLK, H), lambda i: (i, 0))
    pa = pl.BlockSpec((1, _ROW_BLK, H), lambda i: (0, i, 0))
    pb = pl.BlockSpec((1, _ROW_BLK, H), lambda i: (1, i, 0))
    wspec = pl.BlockSpec((H, H), lambda i: (0, 0))
    bspec = pl.BlockSpec((1, H), lambda i: (0, 0))
    row2 = pl.BlockSpec((_ROW_BLK, 2 * H), lambda i: (i, 0))
    out = jax.ShapeDtypeStruct((_NPAD, H), jnp.float32)
    out2 = jax.ShapeDtypeStruct((_NPAD, 2 * H), jnp.float32)
    return pl.pallas_call(
        _combine_norm_proj_body,
        grid=(nblk,),
        in_specs=[row, pa, pb, wspec, bspec, wspec, bspec, wspec, bspec,
                  wspec, bspec],
        out_specs=[row, row2, row],
        out_shape=[out, out2, out],
    )(s1, partials, partials, Wk, bk.reshape(1, H), Wq, bq.reshape(1, H),
      Wv, bv.reshape(1, H), Ws, bs.reshape(1, H))


def _combine_body(s_ref, pa_ref, pb_ref, h_ref):
    h_ref[...] = s_ref[...] + pa_ref[0] + pb_ref[0]


def _combine(s2, partials):
    nblk = N // _ROW_BLK
    row = pl.BlockSpec((_ROW_BLK, H), lambda i: (i, 0))
    pa = pl.BlockSpec((1, _ROW_BLK, H), lambda i: (0, i, 0))
    pb = pl.BlockSpec((1, _ROW_BLK, H), lambda i: (1, i, 0))
    return pl.pallas_call(
        _combine_body,
        grid=(nblk,),
        in_specs=[row, pa, pb],
        out_specs=row,
        out_shape=jax.ShapeDtypeStruct((N, H), jnp.float32),
    )(s2, partials, partials)


def _mlp_body(hs_ref, hd_ref, ft_ref, w1a_ref, w1b_ref, w1c_ref, b1_ref,
              w2_ref, b2_ref, w3_ref, b3_ref, out_ref):
    z = jnp.dot(hs_ref[...], w1a_ref[...])
    z += jnp.dot(hd_ref[...], w1b_ref[...])
    z += jnp.dot(ft_ref[...], w1c_ref[...])
    z += b1_ref[...]
    o = jnp.maximum(z, 0.0)
    o = jnp.maximum(jnp.dot(o, w2_ref[...]) + b2_ref[...], 0.0)
    out_ref[...] = jnp.dot(o, w3_ref[...]) + b3_ref[...]


def _predictor_mlp(hs, hd, ft, Wp1, bp1, Wp2, bp2, Wp3, bp3):
    w1a = Wp1[:H]
    w1b = Wp1[H:2 * H]
    w1c = Wp1[2 * H:]
    nblk = P // _MLP_BLK
    row = pl.BlockSpec((_MLP_BLK, H), lambda i: (i, 0))
    return pl.pallas_call(
        _mlp_body,
        grid=(nblk,),
        in_specs=[
            row, row,
            pl.BlockSpec((_MLP_BLK, 3), lambda i: (i, 0)),
            pl.BlockSpec((H, H), lambda i: (0, 0)),
            pl.BlockSpec((H, H), lambda i: (0, 0)),
            pl.BlockSpec((3, H), lambda i: (0, 0)),
            pl.BlockSpec((1, H), lambda i: (0, 0)),
            pl.BlockSpec((H, H // 2), lambda i: (0, 0)),
            pl.BlockSpec((1, H // 2), lambda i: (0, 0)),
            pl.BlockSpec((H // 2, 1), lambda i: (0, 0)),
            pl.BlockSpec((1, 1), lambda i: (0, 0)),
        ],
        out_specs=pl.BlockSpec((_MLP_BLK, 1), lambda i: (i, 0)),
        out_shape=jax.ShapeDtypeStruct((P, 1), jnp.float32),
    )(hs, hd, ft, w1a, w1b, w1c, bp1.reshape(1, H), Wp2,
      bp2.reshape(1, H // 2), Wp3, bp3.reshape(1, 1))


def _edge_body(kn_hbm, qv_hbm, src_hbm, dst_hbm, zero_hbm, out_hbm,
               srcb, dstb, kd0, qv0, kd1, qv1, shared, sem0, sem1):
    c = lax.axis_index("c")
    s = lax.axis_index("s")
    wid = c * _NS + s
    row0 = s * _ZR
    pltpu.sync_copy(zero_hbm.at[pl.ds(row0, _ZR)],
                    shared.at[pl.ds(row0, _ZR)])

    @pl.when(s == 0)
    def _():
        pltpu.sync_copy(zero_hbm.at[pl.ds(_NS * _ZR, N - _NS * _ZR)],
                        shared.at[pl.ds(_NS * _ZR, N - _NS * _ZR)])

    plsc.subcore_barrier()

    sets = ((kd0, qv0, sem0), (kd1, qv1, sem1))

    def fire(st, j):
        kd, qv, sem = st
        pltpu.async_copy(kn_hbm.at[dstb.at[j]], kd, sem)
        pltpu.async_copy(qv_hbm.at[srcb.at[j]], qv, sem)

    def consume(st, j):
        kd, qv, sem = st
        pltpu.make_async_copy(kn_hbm.at[pl.ds(0, _W)], kd, sem).wait()
        pltpu.make_async_copy(qv_hbm.at[pl.ds(0, _W)], qv, sem).wait()

        @pl.loop(0, _W)
        def _(r):
            for u in range(H // 16):
                sl = pl.ds(u * 16, 16)
                t = jnp.exp(kd[r, sl] - qv[r, sl])
                kd[r, sl] = qv[r, pl.ds(H + u * 16, 16)] / (1.0 + t)

        pltpu.sync_copy(kd, shared.at[dstb.at[j]], add=True)

    @pl.loop(0, _SB)
    def _(sb):
        cr = sb * _KB
        pltpu.sync_copy(src_hbm.at[wid, pl.ds(cr, _KB)], srcb)
        pltpu.sync_copy(dst_hbm.at[wid, pl.ds(cr, _KB)], dstb)
        fire(sets[0], 0)
        for j in range(_KB):
            if j + 1 < _KB:
                fire(sets[(j + 1) % 2], j + 1)
            consume(sets[j % 2], j)

    plsc.subcore_barrier()
    pltpu.sync_copy(shared.at[pl.ds(row0, _ZR)],
                    out_hbm.at[c, pl.ds(row0, _ZR)])

    @pl.when(s == 0)
    def _():
        pltpu.sync_copy(shared.at[pl.ds(_NS * _ZR, N - _NS * _ZR)],
                        out_hbm.at[c, pl.ds(_NS * _ZR, N - _NS * _ZR)])


def _sc_edge_pass(kn, qv, src2, dst2, zeros):
    mesh = plsc.VectorSubcoreMesh(core_axis_name="c", subcore_axis_name="s")
    kfn = pl.kernel(
        _edge_body,
        mesh=mesh,
        out_type=jax.ShapeDtypeStruct((_NC, _NPAD, H), jnp.float32),
        scratch_types=[
            pltpu.VMEM((_KB, _W), jnp.int32),
            pltpu.VMEM((_KB, _W), jnp.int32),
            pltpu.VMEM((_W, H), jnp.float32),
            pltpu.VMEM((_W, 2 * H), jnp.float32),
            pltpu.VMEM((_W, H), jnp.float32),
            pltpu.VMEM((_W, 2 * H), jnp.float32),
            pltpu.VMEM_SHARED((_NPAD, H), jnp.float32),
            pltpu.SemaphoreType.DMA,
            pltpu.SemaphoreType.DMA,
        ],
    )
    return kfn(kn, qv, src2, dst2, zeros)


def _pe_body(h_hbm, pe0_hbm, pe1_hbm, hs_hbm, hd_hbm,
             i0b, i1b, a0, b0, a1, b1, g0, g1, o0, o1):
    c = lax.axis_index("c")
    s = lax.axis_index("s")
    wid = c * _NS + s
    crow0 = wid * _PCHUNKS
    pltpu.sync_copy(pe0_hbm.at[wid], i0b)
    pltpu.sync_copy(pe1_hbm.at[wid], i1b)
    sets = ((a0, b0, g0, o0), (a1, b1, g1, o1))

    def fire(st, j, drain):
        a, b, gs, os = st
        if drain:
            pltpu.make_async_copy(h_hbm.at[pl.ds(0, _PW)], a, os).wait()
            pltpu.make_async_copy(h_hbm.at[pl.ds(0, _PW)], b, os).wait()
        pltpu.async_copy(h_hbm.at[i0b.at[j]], a, gs)
        pltpu.async_copy(h_hbm.at[i1b.at[j]], b, gs)

    def consume(st, j):
        a, b, gs, os = st
        pltpu.make_async_copy(h_hbm.at[pl.ds(0, _PW)], a, gs).wait()
        pltpu.make_async_copy(h_hbm.at[pl.ds(0, _PW)], b, gs).wait()
        off = (crow0 + j) * _PW
        pltpu.async_copy(a, hs_hbm.at[pl.ds(off, _PW)], os)
        pltpu.async_copy(b, hd_hbm.at[pl.ds(off, _PW)], os)

    fire(sets[0], 0, False)
    for j in range(_PCHUNKS):
        if j + 1 < _PCHUNKS:
            fire(sets[(j + 1) % 2], j + 1, drain=(j + 1 >= 2))
        consume(sets[j % 2], j)
    for st in sets:
        _, _, _, os = st
        pltpu.make_async_copy(h_hbm.at[pl.ds(0, _PW)], st[0], os).wait()
        pltpu.make_async_copy(h_hbm.at[pl.ds(0, _PW)], st[1], os).wait()


def _sc_pe_gather(h, pe0_2, pe1_2):
    mesh = plsc.VectorSubcoreMesh(core_axis_name="c", subcore_axis_name="s")
    out = jax.ShapeDtypeStruct((_PPAD, H), jnp.float32)
    kfn = pl.kernel(
        _pe_body,
        mesh=mesh,
        out_type=(out, out),
        scratch_types=[
            pltpu.VMEM((_PCHUNKS, _PW), jnp.int32),
            pltpu.VMEM((_PCHUNKS, _PW), jnp.int32),
            pltpu.VMEM((_PW, H), jnp.float32),
            pltpu.VMEM((_PW, H), jnp.float32),
            pltpu.VMEM((_PW, H), jnp.float32),
            pltpu.VMEM((_PW, H), jnp.float32),
            pltpu.SemaphoreType.DMA,
            pltpu.SemaphoreType.DMA,
            pltpu.SemaphoreType.DMA,
            pltpu.SemaphoreType.DMA,
        ],
    )
    return kfn(h, pe0_2, pe1_2)


def kernel(x, pot_features, Wk1, bk1, Wq1, bq1, Wv1, bv1, Ws1, bs1,
           Wk2, bk2, Wq2, bq2, Wv2, bv2, Ws2, bs2,
           Wp1, bp1, Wp2, bp2, Wp3, bp3, edge_index, pot_edges):
    src2 = jnp.concatenate(
        [edge_index[0], jnp.zeros((_EPAD - E,), jnp.int32)]
    ).reshape(_NW, _ECHUNKS, _W)
    dst2 = jnp.concatenate(
        [edge_index[1], jnp.full((_EPAD - E,), N, jnp.int32)]
    ).reshape(_NW, _ECHUNKS, _W)
    pe0_2 = jnp.concatenate(
        [pot_edges[0], jnp.zeros((_PPAD - P,), jnp.int32)]
    ).reshape(_NW, _PCHUNKS, _PW)
    pe1_2 = jnp.concatenate(
        [pot_edges[1], jnp.zeros((_PPAD - P,), jnp.int32)]
    ).reshape(_NW, _PCHUNKS, _PW)
    zeros = jnp.zeros((_NPAD, H), jnp.float32)

    kn1, qv1, s1 = _proj(x, Wk1, bk1, Wq1, bq1, Wv1, bv1, Ws1, bs1)
    parts1 = _sc_edge_pass(kn1, qv1, src2, dst2, zeros)
    kn2, qv2, s2 = _combine_norm_proj(
        s1, parts1, Wk2, bk2, Wq2, bq2, Wv2, bv2, Ws2, bs2)
    parts2 = _sc_edge_pass(kn2, qv2, src2, dst2, zeros)
    h2 = _combine(s2, parts2)
    hs, hd = _sc_pe_gather(h2, pe0_2, pe1_2)
    return _predictor_mlp(hs, hd, pot_features, Wp1, bp1, Wp2, bp2, Wp3, bp3)


# speedup vs baseline: 1.9718x; 1.9718x over previous
import jax
import jax.numpy as jnp
from jax import lax
from jax.experimental import pallas as pl
from jax.experimental.pallas import tpu as pltpu
from jax.experimental.pallas import tpu_sc as plsc

N = 10000
D = 128
H = 128
P = 100000
E = 320000

_NC = 2
_NS = 16
_NW = _NC * _NS

_W = 64
_KB = 4
_SB = 40
_ECHUNKS = _KB * _SB
_EPAD = _NW * _ECHUNKS * _W

_NPAD = 10008
_ZR = 624

_PW = 128
_PCHUNKS = 26
_PPAD = _NW * _PCHUNKS * _PW

_ROW_BLK = 2000
_MLP_BLK = 2000


def _proj_body(x_ref, wk, bk, wq, bq, wv, bv, ws, bs,
               kn_ref, q_ref, v_ref, s_ref):
    xb = x_ref[...]
    kn_ref[...] = -(jnp.dot(xb, wk[...]) + bk[...])
    q_ref[...] = jnp.dot(xb, wq[...]) + bq[...]
    v_ref[...] = jnp.dot(xb, wv[...]) + bv[...]
    s_ref[...] = jnp.dot(xb, ws[...]) + bs[...]


def _proj(x, Wk, bk, Wq, bq, Wv, bv, Ws, bs):
    nblk = N // _ROW_BLK
    row = pl.BlockSpec((_ROW_BLK, H), lambda i: (i, 0))
    wspec = pl.BlockSpec((H, H), lambda i: (0, 0))
    bspec = pl.BlockSpec((1, H), lambda i: (0, 0))
    out = jax.ShapeDtypeStruct((_NPAD, H), jnp.float32)
    return pl.pallas_call(
        _proj_body,
        grid=(nblk,),
        in_specs=[row, wspec, bspec, wspec, bspec, wspec, bspec, wspec, bspec],
        out_specs=[row, row, row, row],
        out_shape=[out, out, out, out],
    )(x, Wk, bk.reshape(1, H), Wq, bq.reshape(1, H), Wv, bv.reshape(1, H),
      Ws, bs.reshape(1, H))


def _combine_norm_proj_body(s_ref, pa_ref, pb_ref, wk, bk, wq, bq, wv, bv,
                            ws, bs, kn_ref, q_ref, v_ref, s2_ref):
    h = s_ref[...] + pa_ref[0] + pb_ref[0]
    nrm = jnp.sqrt(jnp.sum(h * h, axis=1, keepdims=True))
    h = h / jnp.maximum(nrm, 1e-12)
    h = jnp.maximum(h, 0.0)
    kn_ref[...] = -(jnp.dot(h, wk[...]) + bk[...])
    q_ref[...] = jnp.dot(h, wq[...]) + bq[...]
    v_ref[...] = jnp.dot(h, wv[...]) + bv[...]
    s2_ref[...] = jnp.dot(h, ws[...]) + bs[...]


def _combine_norm_proj(s1, partials, Wk, bk, Wq, bq, Wv, bv, Ws, bs):
    nblk = N // _ROW_BLK
    row = pl.BlockSpec((_ROW_BLK, H), lambda i: (i, 0))
    pa = pl.BlockSpec((1, _ROW_BLK, H), lambda i: (0, i, 0))
    pb = pl.BlockSpec((1, _ROW_BLK, H), lambda i: (1, i, 0))
    wspec = pl.BlockSpec((H, H), lambda i: (0, 0))
    bspec = pl.BlockSpec((1, H), lambda i: (0, 0))
    out = jax.ShapeDtypeStruct((_NPAD, H), jnp.float32)
    return pl.pallas_call(
        _combine_norm_proj_body,
        grid=(nblk,),
        in_specs=[row, pa, pb, wspec, bspec, wspec, bspec, wspec, bspec,
                  wspec, bspec],
        out_specs=[row, row, row, row],
        out_shape=[out, out, out, out],
    )(s1, partials, partials, Wk, bk.reshape(1, H), Wq, bq.reshape(1, H),
      Wv, bv.reshape(1, H), Ws, bs.reshape(1, H))


def _combine_body(s_ref, pa_ref, pb_ref, h_ref):
    h_ref[...] = s_ref[...] + pa_ref[0] + pb_ref[0]


def _combine(s2, partials):
    nblk = N // _ROW_BLK
    row = pl.BlockSpec((_ROW_BLK, H), lambda i: (i, 0))
    pa = pl.BlockSpec((1, _ROW_BLK, H), lambda i: (0, i, 0))
    pb = pl.BlockSpec((1, _ROW_BLK, H), lambda i: (1, i, 0))
    return pl.pallas_call(
        _combine_body,
        grid=(nblk,),
        in_specs=[row, pa, pb],
        out_specs=row,
        out_shape=jax.ShapeDtypeStruct((N, H), jnp.float32),
    )(s2, partials, partials)


def _mlp_body(hs_ref, hd_ref, ft_ref, w1a_ref, w1b_ref, w1c_ref, b1_ref,
              w2_ref, b2_ref, w3_ref, b3_ref, out_ref):
    z = jnp.dot(hs_ref[...], w1a_ref[...])
    z += jnp.dot(hd_ref[...], w1b_ref[...])
    z += jnp.dot(ft_ref[...], w1c_ref[...])
    z += b1_ref[...]
    o = jnp.maximum(z, 0.0)
    o = jnp.maximum(jnp.dot(o, w2_ref[...]) + b2_ref[...], 0.0)
    out_ref[...] = jnp.dot(o, w3_ref[...]) + b3_ref[...]


def _predictor_mlp(hs, hd, ft, Wp1, bp1, Wp2, bp2, Wp3, bp3):
    w1a = Wp1[:H]
    w1b = Wp1[H:2 * H]
    w1c = Wp1[2 * H:]
    nblk = P // _MLP_BLK
    row = pl.BlockSpec((_MLP_BLK, H), lambda i: (i, 0))
    return pl.pallas_call(
        _mlp_body,
        grid=(nblk,),
        in_specs=[
            row, row,
            pl.BlockSpec((_MLP_BLK, 3), lambda i: (i, 0)),
            pl.BlockSpec((H, H), lambda i: (0, 0)),
            pl.BlockSpec((H, H), lambda i: (0, 0)),
            pl.BlockSpec((3, H), lambda i: (0, 0)),
            pl.BlockSpec((1, H), lambda i: (0, 0)),
            pl.BlockSpec((H, H // 2), lambda i: (0, 0)),
            pl.BlockSpec((1, H // 2), lambda i: (0, 0)),
            pl.BlockSpec((H // 2, 1), lambda i: (0, 0)),
            pl.BlockSpec((1, 1), lambda i: (0, 0)),
        ],
        out_specs=pl.BlockSpec((_MLP_BLK, 1), lambda i: (i, 0)),
        out_shape=jax.ShapeDtypeStruct((P, 1), jnp.float32),
    )(hs, hd, ft, w1a, w1b, w1c, bp1.reshape(1, H), Wp2,
      bp2.reshape(1, H // 2), Wp3, bp3.reshape(1, 1))


def _edge_body(kn_hbm, q_hbm, v_hbm, src_hbm, dst_hbm, zero_hbm, out_hbm,
               srcb, dstb, kd0, qs0, vs0, kd1, qs1, vs1, shared, sem0, sem1):
    c = lax.axis_index("c")
    s = lax.axis_index("s")
    wid = c * _NS + s
    row0 = s * _ZR
    pltpu.sync_copy(zero_hbm.at[pl.ds(row0, _ZR)],
                    shared.at[pl.ds(row0, _ZR)])

    @pl.when(s == 0)
    def _():
        pltpu.sync_copy(zero_hbm.at[pl.ds(_NS * _ZR, N - _NS * _ZR)],
                        shared.at[pl.ds(_NS * _ZR, N - _NS * _ZR)])

    plsc.subcore_barrier()

    sets = ((kd0, qs0, vs0, sem0), (kd1, qs1, vs1, sem1))

    def fire(st, j):
        kd, qs, vs, sem = st
        pltpu.async_copy(kn_hbm.at[dstb.at[j]], kd, sem)
        pltpu.async_copy(q_hbm.at[srcb.at[j]], qs, sem)
        pltpu.async_copy(v_hbm.at[srcb.at[j]], vs, sem)

    def consume(st, j):
        kd, qs, vs, sem = st
        for buf in (kd, qs, vs):
            pltpu.make_async_copy(kn_hbm.at[pl.ds(0, _W)], buf, sem).wait()

        @pl.loop(0, _W)
        def _(r):
            for u in range(H // 16):
                sl = pl.ds(u * 16, 16)
                t = jnp.exp(kd[r, sl] - qs[r, sl])
                vs[r, sl] = vs[r, sl] / (1.0 + t)

        pltpu.sync_copy(vs, shared.at[dstb.at[j]], add=True)

    @pl.loop(0, _SB)
    def _(sb):
        cr = sb * _KB
        pltpu.sync_copy(src_hbm.at[wid, pl.ds(cr, _KB)], srcb)
        pltpu.sync_copy(dst_hbm.at[wid, pl.ds(cr, _KB)], dstb)
        fire(sets[0], 0)
        for j in range(_KB):
            if j + 1 < _KB:
                fire(sets[(j + 1) % 2], j + 1)
            consume(sets[j % 2], j)

    plsc.subcore_barrier()
    pltpu.sync_copy(shared.at[pl.ds(row0, _ZR)],
                    out_hbm.at[c, pl.ds(row0, _ZR)])

    @pl.when(s == 0)
    def _():
        pltpu.sync_copy(shared.at[pl.ds(_NS * _ZR, N - _NS * _ZR)],
                        out_hbm.at[c, pl.ds(_NS * _ZR, N - _NS * _ZR)])


def _sc_edge_pass(kn, q, v, src2, dst2, zeros):
    mesh = plsc.VectorSubcoreMesh(core_axis_name="c", subcore_axis_name="s")
    kfn = pl.kernel(
        _edge_body,
        mesh=mesh,
        out_type=jax.ShapeDtypeStruct((_NC, _NPAD, H), jnp.float32),
        scratch_types=[
            pltpu.VMEM((_KB, _W), jnp.int32),
            pltpu.VMEM((_KB, _W), jnp.int32),
            pltpu.VMEM((_W, H), jnp.float32),
            pltpu.VMEM((_W, H), jnp.float32),
            pltpu.VMEM((_W, H), jnp.float32),
            pltpu.VMEM((_W, H), jnp.float32),
            pltpu.VMEM((_W, H), jnp.float32),
            pltpu.VMEM((_W, H), jnp.float32),
            pltpu.VMEM_SHARED((_NPAD, H), jnp.float32),
            pltpu.SemaphoreType.DMA,
            pltpu.SemaphoreType.DMA,
        ],
    )
    return kfn(kn, q, v, src2, dst2, zeros)


def _pe_body(h_hbm, pe0_hbm, pe1_hbm, hs_hbm, hd_hbm,
             i0b, i1b, a0, b0, a1, b1, g0, g1, o0, o1):
    c = lax.axis_index("c")
    s = lax.axis_index("s")
    wid = c * _NS + s
    crow0 = wid * _PCHUNKS
    pltpu.sync_copy(pe0_hbm.at[wid], i0b)
    pltpu.sync_copy(pe1_hbm.at[wid], i1b)
    sets = ((a0, b0, g0, o0), (a1, b1, g1, o1))

    def fire(st, j):
        a, b, gs, _ = st
        pltpu.async_copy(h_hbm.at[i0b.at[j]], a, gs)
        pltpu.async_copy(h_hbm.at[i1b.at[j]], b, gs)

    def consume(st, j):
        a, b, gs, _ = st
        pltpu.make_async_copy(h_hbm.at[pl.ds(0, _PW)], a, gs).wait()
        pltpu.make_async_copy(h_hbm.at[pl.ds(0, _PW)], b, gs).wait()
        off = (crow0 + j) * _PW
        pltpu.sync_copy(a, hs_hbm.at[pl.ds(off, _PW)])
        pltpu.sync_copy(b, hd_hbm.at[pl.ds(off, _PW)])

    fire(sets[0], 0)
    for j in range(_PCHUNKS):
        if j + 1 < _PCHUNKS:
            fire(sets[(j + 1) % 2], j + 1)
        consume(sets[j % 2], j)


def _sc_pe_gather(h, pe0_2, pe1_2):
    mesh = plsc.VectorSubcoreMesh(core_axis_name="c", subcore_axis_name="s")
    out = jax.ShapeDtypeStruct((_PPAD, H), jnp.float32)
    kfn = pl.kernel(
        _pe_body,
        mesh=mesh,
        out_type=(out, out),
        scratch_types=[
            pltpu.VMEM((_PCHUNKS, _PW), jnp.int32),
            pltpu.VMEM((_PCHUNKS, _PW), jnp.int32),
            pltpu.VMEM((_PW, H), jnp.float32),
            pltpu.VMEM((_PW, H), jnp.float32),
            pltpu.VMEM((_PW, H), jnp.float32),
            pltpu.VMEM((_PW, H), jnp.float32),
            pltpu.SemaphoreType.DMA,
            pltpu.SemaphoreType.DMA,
            pltpu.SemaphoreType.DMA,
            pltpu.SemaphoreType.DMA,
        ],
    )
    return kfn(h, pe0_2, pe1_2)


def kernel(x, pot_features, Wk1, bk1, Wq1, bq1, Wv1, bv1, Ws1, bs1,
           Wk2, bk2, Wq2, bq2, Wv2, bv2, Ws2, bs2,
           Wp1, bp1, Wp2, bp2, Wp3, bp3, edge_index, pot_edges):
    src2 = jnp.concatenate(
        [edge_index[0], jnp.zeros((_EPAD - E,), jnp.int32)]
    ).reshape(_NW, _ECHUNKS, _W)
    dst2 = jnp.concatenate(
        [edge_index[1], jnp.full((_EPAD - E,), N, jnp.int32)]
    ).reshape(_NW, _ECHUNKS, _W)
    pe0_2 = jnp.concatenate(
        [pot_edges[0], jnp.zeros((_PPAD - P,), jnp.int32)]
    ).reshape(_NW, _PCHUNKS, _PW)
    pe1_2 = jnp.concatenate(
        [pot_edges[1], jnp.zeros((_PPAD - P,), jnp.int32)]
    ).reshape(_NW, _PCHUNKS, _PW)
    zeros = jnp.zeros((_NPAD, H), jnp.float32)

    kn1, q1, v1, s1 = _proj(x, Wk1, bk1, Wq1, bq1, Wv1, bv1, Ws1, bs1)
    parts1 = _sc_edge_pass(kn1, q1, v1, src2, dst2, zeros)
    kn2, q2, v2, s2 = _combine_norm_proj(
        s1, parts1, Wk2, bk2, Wq2, bq2, Wv2, bv2, Ws2, bs2)
    parts2 = _sc_edge_pass(kn2, q2, v2, src2, dst2, zeros)
    h2 = _combine(s2, parts2)
    hs, hd = _sc_pe_gather(h2, pe0_2, pe1_2)
    return _predictor_mlp(hs, hd, pot_features, Wp1, bp1, Wp2, bp2, Wp3, bp3)


# speedup vs baseline: 2.2091x; 1.1203x over previous
import dataclasses

import jax
import jax.numpy as jnp
from jax import lax
from jax.experimental import pallas as pl
from jax.experimental.pallas import tpu as pltpu
from jax.experimental.pallas import tpu_sc as plsc

N = 10000
D = 128
H = 128
P = 100000
E = 320000

_NC = 2
_NS = 16
_NW = _NC * _NS

_W = 64
_KB = 4
_SB = 40
_ECHUNKS = _KB * _SB
_EPAD = _NW * _ECHUNKS * _W

_NPAD = 10008
_ZR = 624

_PW = 128
_PCHUNKS = 26
_PPAD = _NW * _PCHUNKS * _PW

_ROW_BLK = 2000
_MLP_BLK = 2000


def _pack_bf16(x):
    u = jax.lax.bitcast_convert_type(x, jnp.uint32)
    r = (u + jnp.uint32(0x8000)) >> jnp.uint32(16)
    packed = r[:, :H // 2] | (r[:, H // 2:] << jnp.uint32(16))
    return jax.lax.bitcast_convert_type(packed, jnp.float32)


def _proj_body(x_ref, wk, bk, wq, bq, wv, bv, ws, bs,
               kn_ref, qv_ref, s_ref):
    xb = x_ref[...]
    kn_ref[...] = -(jnp.dot(xb, wk[...]) + bk[...])
    qv_ref[...] = jnp.concatenate(
        [_pack_bf16(jnp.dot(xb, wq[...]) + bq[...]),
         _pack_bf16(jnp.dot(xb, wv[...]) + bv[...])], axis=1)
    s_ref[...] = jnp.dot(xb, ws[...]) + bs[...]


def _proj(x, Wk, bk, Wq, bq, Wv, bv, Ws, bs):
    nblk = N // _ROW_BLK
    row = pl.BlockSpec((_ROW_BLK, H), lambda i: (i, 0))
    wspec = pl.BlockSpec((H, H), lambda i: (0, 0))
    bspec = pl.BlockSpec((1, H), lambda i: (0, 0))
    out = jax.ShapeDtypeStruct((_NPAD, H), jnp.float32)
    return pl.pallas_call(
        _proj_body,
        grid=(nblk,),
        in_specs=[row, wspec, bspec, wspec, bspec, wspec, bspec, wspec, bspec],
        out_specs=[row, row, row],
        out_shape=[out, out, out],
    )(x, Wk, bk.reshape(1, H), Wq, bq.reshape(1, H), Wv, bv.reshape(1, H),
      Ws, bs.reshape(1, H))


def _combine_norm_proj_body(s_ref, pa_ref, pb_ref, wk, bk, wq, bq, wv, bv,
                            ws, bs, kn_ref, qv_ref, s2_ref):
    h = s_ref[...] + pa_ref[0] + pb_ref[0]
    nrm = jnp.sqrt(jnp.sum(h * h, axis=1, keepdims=True))
    h = h / jnp.maximum(nrm, 1e-12)
    h = jnp.maximum(h, 0.0)
    kn_ref[...] = -(jnp.dot(h, wk[...]) + bk[...])
    qv_ref[...] = jnp.concatenate(
        [_pack_bf16(jnp.dot(h, wq[...]) + bq[...]),
         _pack_bf16(jnp.dot(h, wv[...]) + bv[...])], axis=1)
    s2_ref[...] = jnp.dot(h, ws[...]) + bs[...]


def _combine_norm_proj(s1, partials, Wk, bk, Wq, bq, Wv, bv, Ws, bs):
    nblk = N // _ROW_BLK
    row = pl.BlockSpec((_ROW_BLK, H), lambda i: (i, 0))
    pa = pl.BlockSpec((1, _ROW_BLK, H), lambda i: (0, i, 0))
    pb = pl.BlockSpec((1, _ROW_BLK, H), lambda i: (1, i, 0))
    wspec = pl.BlockSpec((H, H), lambda i: (0, 0))
    bspec = pl.BlockSpec((1, H), lambda i: (0, 0))
    out = jax.ShapeDtypeStruct((_NPAD, H), jnp.float32)
    return pl.pallas_call(
        _combine_norm_proj_body,
        grid=(nblk,),
        in_specs=[row, pa, pb, wspec, bspec, wspec, bspec, wspec, bspec,
                  wspec, bspec],
        out_specs=[row, row, row],
        out_shape=[out, out, out],
    )(s1, partials, partials, Wk, bk.reshape(1, H), Wq, bq.reshape(1, H),
      Wv, bv.reshape(1, H), Ws, bs.reshape(1, H))


def _combine_body(s_ref, pa_ref, pb_ref, h_ref):
    h_ref[...] = s_ref[...] + pa_ref[0] + pb_ref[0]


def _combine(s2, partials):
    nblk = N // _ROW_BLK
    row = pl.BlockSpec((_ROW_BLK, H), lambda i: (i, 0))
    pa = pl.BlockSpec((1, _ROW_BLK, H), lambda i: (0, i, 0))
    pb = pl.BlockSpec((1, _ROW_BLK, H), lambda i: (1, i, 0))
    return pl.pallas_call(
        _combine_body,
        grid=(nblk,),
        in_specs=[row, pa, pb],
        out_specs=row,
        out_shape=jax.ShapeDtypeStruct((N, H), jnp.float32),
    )(s2, partials, partials)


def _mlp_body(hs_ref, hd_ref, ft_ref, w1a_ref, w1b_ref, w1c_ref, b1_ref,
              w2_ref, b2_ref, w3_ref, b3_ref, out_ref):
    z = jnp.dot(hs_ref[...], w1a_ref[...])
    z += jnp.dot(hd_ref[...], w1b_ref[...])
    z += jnp.dot(ft_ref[...], w1c_ref[...])
    z += b1_ref[...]
    o = jnp.maximum(z, 0.0)
    o = jnp.maximum(jnp.dot(o, w2_ref[...]) + b2_ref[...], 0.0)
    out_ref[...] = jnp.dot(o, w3_ref[...]) + b3_ref[...]


def _predictor_mlp(hs, hd, ft, Wp1, bp1, Wp2, bp2, Wp3, bp3):
    w1a = Wp1[:H]
    w1b = Wp1[H:2 * H]
    w1c = Wp1[2 * H:]
    nblk = P // _MLP_BLK
    row = pl.BlockSpec((_MLP_BLK, H), lambda i: (i, 0))
    return pl.pallas_call(
        _mlp_body,
        grid=(nblk,),
        in_specs=[
            row, row,
            pl.BlockSpec((_MLP_BLK, 3), lambda i: (i, 0)),
            pl.BlockSpec((H, H), lambda i: (0, 0)),
            pl.BlockSpec((H, H), lambda i: (0, 0)),
            pl.BlockSpec((3, H), lambda i: (0, 0)),
            pl.BlockSpec((1, H), lambda i: (0, 0)),
            pl.BlockSpec((H, H // 2), lambda i: (0, 0)),
            pl.BlockSpec((1, H // 2), lambda i: (0, 0)),
            pl.BlockSpec((H // 2, 1), lambda i: (0, 0)),
            pl.BlockSpec((1, 1), lambda i: (0, 0)),
        ],
        out_specs=pl.BlockSpec((_MLP_BLK, 1), lambda i: (i, 0)),
        out_shape=jax.ShapeDtypeStruct((P, 1), jnp.float32),
    )(hs, hd, ft, w1a, w1b, w1c, bp1.reshape(1, H), Wp2,
      bp2.reshape(1, H // 2), Wp3, bp3.reshape(1, 1))


def _edge_body(kn_hbm, qv_hbm, src_hbm, dst_hbm, zero_hbm, out_hbm,
               srcb, dstb, kd0, qv0, msg0, kd1, qv1, msg1,
               shared, sem0, sem1):
    c = lax.axis_index("c")
    s = lax.axis_index("s")
    wid = c * _NS + s
    row0 = s * _ZR
    pltpu.sync_copy(zero_hbm.at[pl.ds(row0, _ZR)],
                    shared.at[pl.ds(row0, _ZR)])

    @pl.when(s == 0)
    def _():
        pltpu.sync_copy(zero_hbm.at[pl.ds(_NS * _ZR, N - _NS * _ZR)],
                        shared.at[pl.ds(_NS * _ZR, N - _NS * _ZR)])

    plsc.subcore_barrier()

    sets = ((kd0, qv0, msg0, sem0), (kd1, qv1, msg1, sem1))

    def unpk(buf, r, u):
        return plsc.unpack(
            plsc.bitcast(buf[r, pl.ds(u * 16, 16)], jnp.bfloat16),
            format=plsc.PackFormat.INTERLEAVED)

    def fire(st, j):
        kd, qv, msg, sem = st
        pltpu.async_copy(kn_hbm.at[dstb.at[j]], kd, sem)
        pltpu.async_copy(qv_hbm.at[srcb.at[j]], qv, sem)

    def consume(st, j):
        kd, qv, msg, sem = st
        pltpu.make_async_copy(kn_hbm.at[pl.ds(0, _W)], kd, sem).wait()
        pltpu.make_async_copy(kn_hbm.at[pl.ds(0, _W)], qv, sem).wait()

        @pl.loop(0, _W)
        def _(r):
            for u in range(H // 32):
                qa, qb = unpk(qv, r, u)
                va, vb = unpk(qv, r, u + H // 32)
                ka = kd[r, pl.ds(u * 16, 16)]
                kb = kd[r, pl.ds(H // 2 + u * 16, 16)]
                msg[r, pl.ds(u * 16, 16)] = va / (1.0 + jnp.exp(ka - qa))
                msg[r, pl.ds(H // 2 + u * 16, 16)] = (
                    vb / (1.0 + jnp.exp(kb - qb)))

        pltpu.sync_copy(msg, shared.at[dstb.at[j]], add=True)

    @pl.loop(0, _SB)
    def _(sb):
        cr = sb * _KB
        pltpu.sync_copy(src_hbm.at[wid, pl.ds(cr, _KB)], srcb)
        pltpu.sync_copy(dst_hbm.at[wid, pl.ds(cr, _KB)], dstb)
        fire(sets[0], 0)
        for j in range(_KB):
            if j + 1 < _KB:
                fire(sets[(j + 1) % 2], j + 1)
            consume(sets[j % 2], j)

    plsc.subcore_barrier()
    pltpu.sync_copy(shared.at[pl.ds(row0, _ZR)],
                    out_hbm.at[c, pl.ds(row0, _ZR)])

    @pl.when(s == 0)
    def _():
        pltpu.sync_copy(shared.at[pl.ds(_NS * _ZR, N - _NS * _ZR)],
                        out_hbm.at[c, pl.ds(_NS * _ZR, N - _NS * _ZR)])


def _sc_compiler_params():
    cp = pltpu.CompilerParams()
    if "needs_layout_passes" in pltpu.CompilerParams.__dataclass_fields__:
        cp = dataclasses.replace(cp, needs_layout_passes=False)
    return cp


def _sc_edge_pass(kn, qv, src2, dst2, zeros):
    mesh = plsc.VectorSubcoreMesh(core_axis_name="c", subcore_axis_name="s")
    kfn = pl.kernel(
        _edge_body,
        mesh=mesh,
        compiler_params=_sc_compiler_params(),
        out_type=jax.ShapeDtypeStruct((_NC, _NPAD, H), jnp.float32),
        scratch_types=[
            pltpu.VMEM((_KB, _W), jnp.int32),
            pltpu.VMEM((_KB, _W), jnp.int32),
            pltpu.VMEM((_W, H), jnp.float32),
            pltpu.VMEM((_W, H), jnp.float32),
            pltpu.VMEM((_W, H), jnp.float32),
            pltpu.VMEM((_W, H), jnp.float32),
            pltpu.VMEM((_W, H), jnp.float32),
            pltpu.VMEM((_W, H), jnp.float32),
            pltpu.VMEM_SHARED((_NPAD, H), jnp.float32),
            pltpu.SemaphoreType.DMA,
            pltpu.SemaphoreType.DMA,
        ],
    )
    return kfn(kn, qv, src2, dst2, zeros)


def _pe_body(h_hbm, pe0_hbm, pe1_hbm, hs_hbm, hd_hbm,
             i0b, i1b, a0, b0, a1, b1, g0, g1, o0, o1):
    c = lax.axis_index("c")
    s = lax.axis_index("s")
    wid = c * _NS + s
    crow0 = wid * _PCHUNKS
    pltpu.sync_copy(pe0_hbm.at[wid], i0b)
    pltpu.sync_copy(pe1_hbm.at[wid], i1b)
    sets = ((a0, b0, g0, o0), (a1, b1, g1, o1))

    def fire(st, j):
        a, b, gs, _ = st
        pltpu.async_copy(h_hbm.at[i0b.at[j]], a, gs)
        pltpu.async_copy(h_hbm.at[i1b.at[j]], b, gs)

    def consume(st, j):
        a, b, gs, _ = st
        pltpu.make_async_copy(h_hbm.at[pl.ds(0, _PW)], a, gs).wait()
        pltpu.make_async_copy(h_hbm.at[pl.ds(0, _PW)], b, gs).wait()
        off = (crow0 + j) * _PW
        pltpu.sync_copy(a, hs_hbm.at[pl.ds(off, _PW)])
        pltpu.sync_copy(b, hd_hbm.at[pl.ds(off, _PW)])

    fire(sets[0], 0)
    for j in range(_PCHUNKS):
        if j + 1 < _PCHUNKS:
            fire(sets[(j + 1) % 2], j + 1)
        consume(sets[j % 2], j)


def _sc_pe_gather(h, pe0_2, pe1_2):
    mesh = plsc.VectorSubcoreMesh(core_axis_name="c", subcore_axis_name="s")
    out = jax.ShapeDtypeStruct((_PPAD, H), jnp.float32)
    kfn = pl.kernel(
        _pe_body,
        mesh=mesh,
        out_type=(out, out),
        scratch_types=[
            pltpu.VMEM((_PCHUNKS, _PW), jnp.int32),
            pltpu.VMEM((_PCHUNKS, _PW), jnp.int32),
            pltpu.VMEM((_PW, H), jnp.float32),
            pltpu.VMEM((_PW, H), jnp.float32),
            pltpu.VMEM((_PW, H), jnp.float32),
            pltpu.VMEM((_PW, H), jnp.float32),
            pltpu.SemaphoreType.DMA,
            pltpu.SemaphoreType.DMA,
            pltpu.SemaphoreType.DMA,
            pltpu.SemaphoreType.DMA,
        ],
    )
    return kfn(h, pe0_2, pe1_2)


def kernel(x, pot_features, Wk1, bk1, Wq1, bq1, Wv1, bv1, Ws1, bs1,
           Wk2, bk2, Wq2, bq2, Wv2, bv2, Ws2, bs2,
           Wp1, bp1, Wp2, bp2, Wp3, bp3, edge_index, pot_edges):
    src2 = jnp.concatenate(
        [edge_index[0], jnp.zeros((_EPAD - E,), jnp.int32)]
    ).reshape(_NW, _ECHUNKS, _W)
    dst2 = jnp.concatenate(
        [edge_index[1], jnp.full((_EPAD - E,), N, jnp.int32)]
    ).reshape(_NW, _ECHUNKS, _W)
    pe0_2 = jnp.concatenate(
        [pot_edges[0], jnp.zeros((_PPAD - P,), jnp.int32)]
    ).reshape(_NW, _PCHUNKS, _PW)
    pe1_2 = jnp.concatenate(
        [pot_edges[1], jnp.zeros((_PPAD - P,), jnp.int32)]
    ).reshape(_NW, _PCHUNKS, _PW)
    zeros = jnp.zeros((_NPAD, H), jnp.float32)

    kn1, qv1, s1 = _proj(x, Wk1, bk1, Wq1, bq1, Wv1, bv1, Ws1, bs1)
    parts1 = _sc_edge_pass(kn1, qv1, src2, dst2, zeros)
    kn2, qv2, s2 = _combine_norm_proj(
        s1, parts1, Wk2, bk2, Wq2, bq2, Wv2, bv2, Ws2, bs2)
    parts2 = _sc_edge_pass(kn2, qv2, src2, dst2, zeros)
    h2 = _combine(s2, parts2)
    hs, hd = _sc_pe_gather(h2, pe0_2, pe1_2)
    return _predictor_mlp(hs, hd, pot_features, Wp1, bp1, Wp2, bp2, Wp3, bp3)


# speedup vs baseline: 2.4277x; 1.0989x over previous
import dataclasses

import jax
import jax.numpy as jnp
from jax import lax
from jax.experimental import pallas as pl
from jax.experimental.pallas import tpu as pltpu
from jax.experimental.pallas import tpu_sc as plsc

N = 10000
D = 128
H = 128
P = 100000
E = 320000

_NC = 2
_NS = 16
_NW = _NC * _NS

_W = 40
_KB = 8
_SB = 32
_ECHUNKS = _KB * _SB
_EPAD = _NW * _ECHUNKS * _W

_NPAD = 10008
_ZR = 624

_PW = 128
_PCHUNKS = 26
_PPAD = _NW * _PCHUNKS * _PW

_ROW_BLK = 2000
_MLP_BLK = 2000


def _pack_bf16(x):
    u = jax.lax.bitcast_convert_type(x, jnp.uint32)
    r = (u + jnp.uint32(0x8000)) >> jnp.uint32(16)
    packed = r[:, :H // 2] | (r[:, H // 2:] << jnp.uint32(16))
    return jax.lax.bitcast_convert_type(packed, jnp.float32)


def _proj_body(x_ref, wk, bk, wq, bq, wv, bv, ws, bs,
               kn_ref, qv_ref, s_ref):
    xb = x_ref[...]
    kn_ref[...] = -(jnp.dot(xb, wk[...]) + bk[...])
    qv_ref[...] = jnp.concatenate(
        [_pack_bf16(jnp.dot(xb, wq[...]) + bq[...]),
         _pack_bf16(jnp.dot(xb, wv[...]) + bv[...])], axis=1)
    s_ref[...] = jnp.dot(xb, ws[...]) + bs[...]


def _proj(x, Wk, bk, Wq, bq, Wv, bv, Ws, bs):
    nblk = N // _ROW_BLK
    row = pl.BlockSpec((_ROW_BLK, H), lambda i: (i, 0))
    wspec = pl.BlockSpec((H, H), lambda i: (0, 0))
    bspec = pl.BlockSpec((1, H), lambda i: (0, 0))
    out = jax.ShapeDtypeStruct((_NPAD, H), jnp.float32)
    return pl.pallas_call(
        _proj_body,
        grid=(nblk,),
        in_specs=[row, wspec, bspec, wspec, bspec, wspec, bspec, wspec, bspec],
        out_specs=[row, row, row],
        out_shape=[out, out, out],
    )(x, Wk, bk.reshape(1, H), Wq, bq.reshape(1, H), Wv, bv.reshape(1, H),
      Ws, bs.reshape(1, H))


def _combine_norm_proj_body(s_ref, pa_ref, pb_ref, wk, bk, wq, bq, wv, bv,
                            ws, bs, kn_ref, qv_ref, s2_ref):
    h = s_ref[...] + pa_ref[0] + pb_ref[0]
    nrm = jnp.sqrt(jnp.sum(h * h, axis=1, keepdims=True))
    h = h / jnp.maximum(nrm, 1e-12)
    h = jnp.maximum(h, 0.0)
    kn_ref[...] = -(jnp.dot(h, wk[...]) + bk[...])
    qv_ref[...] = jnp.concatenate(
        [_pack_bf16(jnp.dot(h, wq[...]) + bq[...]),
         _pack_bf16(jnp.dot(h, wv[...]) + bv[...])], axis=1)
    s2_ref[...] = jnp.dot(h, ws[...]) + bs[...]


def _combine_norm_proj(s1, partials, Wk, bk, Wq, bq, Wv, bv, Ws, bs):
    nblk = N // _ROW_BLK
    row = pl.BlockSpec((_ROW_BLK, H), lambda i: (i, 0))
    pa = pl.BlockSpec((1, _ROW_BLK, H), lambda i: (0, i, 0))
    pb = pl.BlockSpec((1, _ROW_BLK, H), lambda i: (1, i, 0))
    wspec = pl.BlockSpec((H, H), lambda i: (0, 0))
    bspec = pl.BlockSpec((1, H), lambda i: (0, 0))
    out = jax.ShapeDtypeStruct((_NPAD, H), jnp.float32)
    return pl.pallas_call(
        _combine_norm_proj_body,
        grid=(nblk,),
        in_specs=[row, pa, pb, wspec, bspec, wspec, bspec, wspec, bspec,
                  wspec, bspec],
        out_specs=[row, row, row],
        out_shape=[out, out, out],
    )(s1, partials, partials, Wk, bk.reshape(1, H), Wq, bq.reshape(1, H),
      Wv, bv.reshape(1, H), Ws, bs.reshape(1, H))


def _combine_body(s_ref, pa_ref, pb_ref, h_ref):
    h_ref[...] = s_ref[...] + pa_ref[0] + pb_ref[0]


def _combine(s2, partials):
    nblk = N // _ROW_BLK
    row = pl.BlockSpec((_ROW_BLK, H), lambda i: (i, 0))
    pa = pl.BlockSpec((1, _ROW_BLK, H), lambda i: (0, i, 0))
    pb = pl.BlockSpec((1, _ROW_BLK, H), lambda i: (1, i, 0))
    return pl.pallas_call(
        _combine_body,
        grid=(nblk,),
        in_specs=[row, pa, pb],
        out_specs=row,
        out_shape=jax.ShapeDtypeStruct((N, H), jnp.float32),
    )(s2, partials, partials)


def _mlp_body(hs_ref, hd_ref, ft_ref, w1a_ref, w1b_ref, w1c_ref, b1_ref,
              w2_ref, b2_ref, w3_ref, b3_ref, out_ref):
    z = jnp.dot(hs_ref[...], w1a_ref[...])
    z += jnp.dot(hd_ref[...], w1b_ref[...])
    z += jnp.dot(ft_ref[...], w1c_ref[...])
    z += b1_ref[...]
    o = jnp.maximum(z, 0.0)
    o = jnp.maximum(jnp.dot(o, w2_ref[...]) + b2_ref[...], 0.0)
    out_ref[...] = jnp.dot(o, w3_ref[...]) + b3_ref[...]


def _predictor_mlp(hs, hd, ft, Wp1, bp1, Wp2, bp2, Wp3, bp3):
    w1a = Wp1[:H]
    w1b = Wp1[H:2 * H]
    w1c = Wp1[2 * H:]
    nblk = P // _MLP_BLK
    row = pl.BlockSpec((_MLP_BLK, H), lambda i: (i, 0))
    return pl.pallas_call(
        _mlp_body,
        grid=(nblk,),
        in_specs=[
            row, row,
            pl.BlockSpec((_MLP_BLK, 3), lambda i: (i, 0)),
            pl.BlockSpec((H, H), lambda i: (0, 0)),
            pl.BlockSpec((H, H), lambda i: (0, 0)),
            pl.BlockSpec((3, H), lambda i: (0, 0)),
            pl.BlockSpec((1, H), lambda i: (0, 0)),
            pl.BlockSpec((H, H // 2), lambda i: (0, 0)),
            pl.BlockSpec((1, H // 2), lambda i: (0, 0)),
            pl.BlockSpec((H // 2, 1), lambda i: (0, 0)),
            pl.BlockSpec((1, 1), lambda i: (0, 0)),
        ],
        out_specs=pl.BlockSpec((_MLP_BLK, 1), lambda i: (i, 0)),
        out_shape=jax.ShapeDtypeStruct((P, 1), jnp.float32),
    )(hs, hd, ft, w1a, w1b, w1c, bp1.reshape(1, H), Wp2,
      bp2.reshape(1, H // 2), Wp3, bp3.reshape(1, 1))


def _edge_body(kn_hbm, qv_hbm, src_hbm, dst_hbm, zero_hbm, out_hbm,
               srcb, dstb, kd0, qv0, msg0, kd1, qv1, msg1, kd2, qv2, msg2,
               shared, sem0, sem1, sem2):
    c = lax.axis_index("c")
    s = lax.axis_index("s")
    wid = c * _NS + s
    row0 = s * _ZR
    pltpu.sync_copy(zero_hbm.at[pl.ds(row0, _ZR)],
                    shared.at[pl.ds(row0, _ZR)])

    @pl.when(s == 0)
    def _():
        pltpu.sync_copy(zero_hbm.at[pl.ds(_NS * _ZR, N - _NS * _ZR)],
                        shared.at[pl.ds(_NS * _ZR, N - _NS * _ZR)])

    plsc.subcore_barrier()

    sets = ((kd0, qv0, msg0, sem0), (kd1, qv1, msg1, sem1),
            (kd2, qv2, msg2, sem2))

    def unpk(buf, r, u):
        return plsc.unpack(
            plsc.bitcast(buf[r, pl.ds(u * 16, 16)], jnp.bfloat16),
            format=plsc.PackFormat.INTERLEAVED)

    def fire(st, j):
        kd, qv, msg, sem = st
        pltpu.async_copy(kn_hbm.at[dstb.at[j]], kd, sem)
        pltpu.async_copy(qv_hbm.at[srcb.at[j]], qv, sem)

    def consume(st, j):
        kd, qv, msg, sem = st
        pltpu.make_async_copy(kn_hbm.at[pl.ds(0, _W)], kd, sem).wait()
        pltpu.make_async_copy(qv_hbm.at[pl.ds(0, _W)], qv, sem).wait()

        @pl.loop(0, _W)
        def _(r):
            for u in range(H // 32):
                qa, qb = unpk(qv, r, u)
                va, vb = unpk(qv, r, u + H // 32)
                ka = kd[r, pl.ds(u * 16, 16)]
                kb = kd[r, pl.ds(H // 2 + u * 16, 16)]
                msg[r, pl.ds(u * 16, 16)] = va / (1.0 + jnp.exp(ka - qa))
                msg[r, pl.ds(H // 2 + u * 16, 16)] = (
                    vb / (1.0 + jnp.exp(kb - qb)))

        pltpu.sync_copy(msg, shared.at[dstb.at[j]], add=True)

    @pl.loop(0, _SB)
    def _(sb):
        cr = sb * _KB
        pltpu.sync_copy(src_hbm.at[wid, pl.ds(cr, _KB)], srcb)
        pltpu.sync_copy(dst_hbm.at[wid, pl.ds(cr, _KB)], dstb)
        fire(sets[0], 0)
        fire(sets[1], 1)
        for j in range(_KB):
            if j + 2 < _KB:
                fire(sets[(j + 2) % 3], j + 2)
            consume(sets[j % 3], j)

    plsc.subcore_barrier()
    pltpu.sync_copy(shared.at[pl.ds(row0, _ZR)],
                    out_hbm.at[c, pl.ds(row0, _ZR)])

    @pl.when(s == 0)
    def _():
        pltpu.sync_copy(shared.at[pl.ds(_NS * _ZR, N - _NS * _ZR)],
                        out_hbm.at[c, pl.ds(_NS * _ZR, N - _NS * _ZR)])


def _sc_compiler_params():
    cp = pltpu.CompilerParams()
    if "needs_layout_passes" in pltpu.CompilerParams.__dataclass_fields__:
        cp = dataclasses.replace(cp, needs_layout_passes=False)
    return cp


def _sc_edge_pass(kn, qv, src2, dst2, zeros):
    mesh = plsc.VectorSubcoreMesh(core_axis_name="c", subcore_axis_name="s")
    kfn = pl.kernel(
        _edge_body,
        mesh=mesh,
        compiler_params=_sc_compiler_params(),
        out_type=jax.ShapeDtypeStruct((_NC, _NPAD, H), jnp.float32),
        scratch_types=[
            pltpu.VMEM((_KB, _W), jnp.int32),
            pltpu.VMEM((_KB, _W), jnp.int32),
            pltpu.VMEM((_W, H), jnp.float32),
            pltpu.VMEM((_W, H), jnp.float32),
            pltpu.VMEM((_W, H), jnp.float32),
            pltpu.VMEM((_W, H), jnp.float32),
            pltpu.VMEM((_W, H), jnp.float32),
            pltpu.VMEM((_W, H), jnp.float32),
            pltpu.VMEM((_W, H), jnp.float32),
            pltpu.VMEM((_W, H), jnp.float32),
            pltpu.VMEM((_W, H), jnp.float32),
            pltpu.VMEM_SHARED((_NPAD, H), jnp.float32),
            pltpu.SemaphoreType.DMA,
            pltpu.SemaphoreType.DMA,
            pltpu.SemaphoreType.DMA,
        ],
    )
    return kfn(kn, qv, src2, dst2, zeros)


def _pe_body(h_hbm, pe0_hbm, pe1_hbm, hs_hbm, hd_hbm,
             i0b, i1b, a0, b0, a1, b1, g0, g1, o0, o1):
    c = lax.axis_index("c")
    s = lax.axis_index("s")
    wid = c * _NS + s
    crow0 = wid * _PCHUNKS
    pltpu.sync_copy(pe0_hbm.at[wid], i0b)
    pltpu.sync_copy(pe1_hbm.at[wid], i1b)
    sets = ((a0, b0, g0, o0), (a1, b1, g1, o1))

    def fire(st, j):
        a, b, gs, _ = st
        pltpu.async_copy(h_hbm.at[i0b.at[j]], a, gs)
        pltpu.async_copy(h_hbm.at[i1b.at[j]], b, gs)

    def consume(st, j):
        a, b, gs, _ = st
        pltpu.make_async_copy(h_hbm.at[pl.ds(0, _PW)], a, gs).wait()
        pltpu.make_async_copy(h_hbm.at[pl.ds(0, _PW)], b, gs).wait()
        off = (crow0 + j) * _PW
        pltpu.sync_copy(a, hs_hbm.at[pl.ds(off, _PW)])
        pltpu.sync_copy(b, hd_hbm.at[pl.ds(off, _PW)])

    fire(sets[0], 0)
    for j in range(_PCHUNKS):
        if j + 1 < _PCHUNKS:
            fire(sets[(j + 1) % 2], j + 1)
        consume(sets[j % 2], j)


def _sc_pe_gather(h, pe0_2, pe1_2):
    mesh = plsc.VectorSubcoreMesh(core_axis_name="c", subcore_axis_name="s")
    out = jax.ShapeDtypeStruct((_PPAD, H), jnp.float32)
    kfn = pl.kernel(
        _pe_body,
        mesh=mesh,
        out_type=(out, out),
        scratch_types=[
            pltpu.VMEM((_PCHUNKS, _PW), jnp.int32),
            pltpu.VMEM((_PCHUNKS, _PW), jnp.int32),
            pltpu.VMEM((_PW, H), jnp.float32),
            pltpu.VMEM((_PW, H), jnp.float32),
            pltpu.VMEM((_PW, H), jnp.float32),
            pltpu.VMEM((_PW, H), jnp.float32),
            pltpu.SemaphoreType.DMA,
            pltpu.SemaphoreType.DMA,
            pltpu.SemaphoreType.DMA,
            pltpu.SemaphoreType.DMA,
        ],
    )
    return kfn(h, pe0_2, pe1_2)


def kernel(x, pot_features, Wk1, bk1, Wq1, bq1, Wv1, bv1, Ws1, bs1,
           Wk2, bk2, Wq2, bq2, Wv2, bv2, Ws2, bs2,
           Wp1, bp1, Wp2, bp2, Wp3, bp3, edge_index, pot_edges):
    src2 = jnp.concatenate(
        [edge_index[0], jnp.zeros((_EPAD - E,), jnp.int32)]
    ).reshape(_NW, _ECHUNKS, _W)
    dst2 = jnp.concatenate(
        [edge_index[1], jnp.full((_EPAD - E,), N, jnp.int32)]
    ).reshape(_NW, _ECHUNKS, _W)
    pe0_2 = jnp.concatenate(
        [pot_edges[0], jnp.zeros((_PPAD - P,), jnp.int32)]
    ).reshape(_NW, _PCHUNKS, _PW)
    pe1_2 = jnp.concatenate(
        [pot_edges[1], jnp.zeros((_PPAD - P,), jnp.int32)]
    ).reshape(_NW, _PCHUNKS, _PW)
    zeros = jnp.zeros((_NPAD, H), jnp.float32)

    kn1, qv1, s1 = _proj(x, Wk1, bk1, Wq1, bq1, Wv1, bv1, Ws1, bs1)
    parts1 = _sc_edge_pass(kn1, qv1, src2, dst2, zeros)
    kn2, qv2, s2 = _combine_norm_proj(
        s1, parts1, Wk2, bk2, Wq2, bq2, Wv2, bv2, Ws2, bs2)
    parts2 = _sc_edge_pass(kn2, qv2, src2, dst2, zeros)
    h2 = _combine(s2, parts2)
    hs, hd = _sc_pe_gather(h2, pe0_2, pe1_2)
    return _predictor_mlp(hs, hd, pot_features, Wp1, bp1, Wp2, bp2, Wp3, bp3)
